# baseline (speedup 1.0000x reference)
import jax
import jax.numpy as jnp
from jax import lax
from jax.experimental import pallas as pl
from jax.experimental.pallas import tpu as pltpu
from jax.experimental.pallas import tpu_sc as plsc

D = 64
SCALE = 8.0
NC = 2
NS = 16
NW = NC * NS
LANES = 16

V = 1000000
CB = 512
NBLK = V // CB
VTAIL = V - NBLK * CB

NBUF = 4
LOOKAHEAD = 2
C = 256


def _transpose_scale(tab_t, tail_lin):
    mesh = plsc.VectorSubcoreMesh(core_axis_name="c", subcore_axis_name="s")

    @pl.kernel(
        out_type=jax.ShapeDtypeStruct((V * D,), jnp.float32),
        mesh=mesh,
        scratch_types=[
            pltpu.VMEM((D, CB), jnp.float32),
            pltpu.VMEM((CB * D,), jnp.float32),
            pltpu.VMEM((VTAIL * D,), jnp.float32),
        ],
        compiler_params=pltpu.CompilerParams(
            use_tc_tiling_on_sc=True, needs_layout_passes=False),
    )
    def tkern(tab_hbm, tail_hbm, out_hbm, vbuf, obuf, tbuf):
        wid = lax.axis_index("s") * NC + lax.axis_index("c")
        n_w = jnp.where(wid < NBLK % NW, NBLK // NW + 1, NBLK // NW)

        lane = lax.iota(jnp.int32, 16)

        def blk_body(t, carry):
            c0 = (wid + t * NW) * CB
            c0 = pl.multiple_of(c0, 128)
            pltpu.sync_copy(tab_hbm.at[:, pl.ds(c0, CB)], vbuf)
            for d in range(D):
                @plsc.parallel_loop(0, CB // LANES, 1, unroll=4)
                def _(k):
                    vals = vbuf[d, pl.ds(k * LANES, LANES)] * SCALE
                    addr = (k * LANES + lane) * D + d
                    plsc.store_scatter(obuf, [addr], vals)
            pltpu.sync_copy(obuf, out_hbm.at[pl.ds(c0 * D, CB * D)])
            return carry

        lax.fori_loop(0, n_w, blk_body, 0)

        @pl.when(wid == NW - 1)
        def _():
            pltpu.sync_copy(tail_hbm, tbuf)
            pltpu.sync_copy(tbuf, out_hbm.at[pl.ds(NBLK * CB * D, VTAIL * D)])

    return tkern(tab_t, tail_lin)


def _gather(idx_flat, table_lin):
    B = idx_flat.shape[0]
    per_w = B // NW
    n_chunks = per_w // C
    n_groups = n_chunks // NBUF

    mesh = plsc.VectorSubcoreMesh(core_axis_name="c", subcore_axis_name="s")

    @pl.kernel(
        out_type=jax.ShapeDtypeStruct((B, D), jnp.float32),
        mesh=mesh,
        scratch_types=(
            [pltpu.VMEM((per_w,), jnp.int32)]
            + [pltpu.VMEM((C, D), jnp.float32) for _ in range(NBUF)]
            + [pltpu.SemaphoreType.DMA for _ in range(NBUF)]
            + [pltpu.SemaphoreType.DMA for _ in range(NBUF)]
        ),
        compiler_params=pltpu.CompilerParams(use_tc_tiling_on_sc=False),
    )
    def gkern(idx_hbm, table_hbm, out_hbm, idx_v, *bufs_and_sems):
        bufs = bufs_and_sems[:NBUF]
        gsem = bufs_and_sems[NBUF:2 * NBUF]
        ssem = bufs_and_sems[2 * NBUF:3 * NBUF]

        wid = lax.axis_index("s") * NC + lax.axis_index("c")
        base = wid * per_w

        pltpu.sync_copy(idx_hbm.at[pl.ds(base, per_w)], idx_v)

        def issue_gather(g, b):
            pltpu.async_copy(
                table_hbm.at[idx_v.at[pl.ds(g * C, C)]], bufs[b], gsem[b])

        def wait_gather(g, b):
            pltpu.make_async_copy(
                table_hbm.at[idx_v.at[pl.ds(g * C, C)]], bufs[b],
                gsem[b]).wait()

        def issue_store(g, b):
            pltpu.async_copy(
                bufs[b], out_hbm.at[pl.ds(base + g * C, C)], ssem[b])

        def wait_store(g, b):
            pltpu.make_async_copy(
                bufs[b], out_hbm.at[pl.ds(base + g * C, C)], ssem[b]).wait()

        for g in range(LOOKAHEAD):
            issue_gather(g, g)

        def group_body(gi, carry):
            for p in range(NBUF):
                g = gi * NBUF + p
                q = (p + LOOKAHEAD) % NBUF
                wait_gather(g, p)
                issue_store(g, p)
                if p < NBUF - LOOKAHEAD:
                    @pl.when(gi >= 1)
                    def _():
                        wait_store(g - LOOKAHEAD, q)
                    issue_gather(g + LOOKAHEAD, q)
                else:
                    @pl.when(gi < n_groups - 1)
                    def _():
                        wait_store(g - LOOKAHEAD, q)
                        issue_gather(g + LOOKAHEAD, q)
            return carry

        lax.fori_loop(0, n_groups, group_body, 0)

        for p in range(NBUF):
            wait_store(n_chunks - NBUF + p, p)

    return gkern(idx_flat, table_lin)


def kernel(indices, table):
    B = indices.shape[0] * indices.shape[1]
    idx_flat = indices.reshape(B).astype(jnp.int32)

    tail_lin = (table[NBLK * CB:] * SCALE).reshape(VTAIL * D)
    tab_lin = _transpose_scale(table.T, tail_lin)
    out = _gather(idx_flat, tab_lin.reshape(V, D))
    return out.reshape(indices.shape[0], indices.shape[1], D)


# speedup vs baseline: 1.1799x; 1.1799x over previous
import jax
import jax.numpy as jnp
from jax import lax
from jax.experimental import pallas as pl
from jax.experimental.pallas import tpu as pltpu
from jax.experimental.pallas import tpu_sc as plsc

D = 64
SCALE = 8.0
NC = 2
NS = 16
NW = NC * NS
LANES = 16

V = 1000000
CB = 896
NBLK = V // CB
VTAIL = V - NBLK * CB

NBUF = 4
LOOKAHEAD = 2
C = 256


def _transpose_scale(tab_t, tail_lin):
    mesh = plsc.VectorSubcoreMesh(core_axis_name="c", subcore_axis_name="s")

    @pl.kernel(
        out_type=jax.ShapeDtypeStruct((V * D,), jnp.float32),
        mesh=mesh,
        scratch_types=[
            pltpu.VMEM((D, CB), jnp.float32),
            pltpu.VMEM((CB * D,), jnp.float32),
            pltpu.VMEM((VTAIL * D,), jnp.float32),
        ],
        compiler_params=pltpu.CompilerParams(
            use_tc_tiling_on_sc=True, needs_layout_passes=False),
    )
    def tkern(tab_hbm, tail_hbm, out_hbm, vbuf, obuf, tbuf):
        wid = lax.axis_index("s") * NC + lax.axis_index("c")
        n_w = jnp.where(wid < NBLK % NW, NBLK // NW + 1, NBLK // NW)

        lane = lax.iota(jnp.int32, 16)

        def blk_body(t, carry):
            c0 = (wid + t * NW) * CB
            c0 = pl.multiple_of(c0, 128)
            pltpu.sync_copy(tab_hbm.at[:, pl.ds(c0, CB)], vbuf)

            @plsc.parallel_loop(0, D * CB // LANES, 1, unroll=8)
            def _(t):
                d = t & (D - 1)
                k = t >> 6
                row = jnp.broadcast_to(d, (LANES,))
                vals = plsc.load_gather(vbuf, [row, k * LANES + lane]) * SCALE
                addr = (k * LANES + lane) * D + d
                plsc.store_scatter(obuf, [addr], vals)

            pltpu.sync_copy(obuf, out_hbm.at[pl.ds(c0 * D, CB * D)])
            return carry

        lax.fori_loop(0, n_w, blk_body, 0)

        @pl.when(wid == NW - 1)
        def _():
            pltpu.sync_copy(tail_hbm, tbuf)
            pltpu.sync_copy(tbuf, out_hbm.at[pl.ds(NBLK * CB * D, VTAIL * D)])

    return tkern(tab_t, tail_lin)


def _gather(idx_flat, table_lin):
    B = idx_flat.shape[0]
    per_w = B // NW
    n_chunks = per_w // C
    n_groups = n_chunks // NBUF

    mesh = plsc.VectorSubcoreMesh(core_axis_name="c", subcore_axis_name="s")

    @pl.kernel(
        out_type=jax.ShapeDtypeStruct((B, D), jnp.float32),
        mesh=mesh,
        scratch_types=(
            [pltpu.VMEM((per_w,), jnp.int32)]
            + [pltpu.VMEM((C, D), jnp.float32) for _ in range(NBUF)]
            + [pltpu.SemaphoreType.DMA for _ in range(NBUF)]
            + [pltpu.SemaphoreType.DMA for _ in range(NBUF)]
        ),
        compiler_params=pltpu.CompilerParams(use_tc_tiling_on_sc=False),
    )
    def gkern(idx_hbm, table_hbm, out_hbm, idx_v, *bufs_and_sems):
        bufs = bufs_and_sems[:NBUF]
        gsem = bufs_and_sems[NBUF:2 * NBUF]
        ssem = bufs_and_sems[2 * NBUF:3 * NBUF]

        wid = lax.axis_index("s") * NC + lax.axis_index("c")
        base = wid * per_w

        pltpu.sync_copy(idx_hbm.at[pl.ds(base, per_w)], idx_v)

        def issue_gather(g, b):
            pltpu.async_copy(
                table_hbm.at[idx_v.at[pl.ds(g * C, C)]], bufs[b], gsem[b])

        def wait_gather(g, b):
            pltpu.make_async_copy(
                table_hbm.at[idx_v.at[pl.ds(g * C, C)]], bufs[b],
                gsem[b]).wait()

        def issue_store(g, b):
            pltpu.async_copy(
                bufs[b], out_hbm.at[pl.ds(base + g * C, C)], ssem[b])

        def wait_store(g, b):
            pltpu.make_async_copy(
                bufs[b], out_hbm.at[pl.ds(base + g * C, C)], ssem[b]).wait()

        for g in range(LOOKAHEAD):
            issue_gather(g, g)

        def group_body(gi, carry):
            for p in range(NBUF):
                g = gi * NBUF + p
                q = (p + LOOKAHEAD) % NBUF
                wait_gather(g, p)
                issue_store(g, p)
                if p < NBUF - LOOKAHEAD:
                    @pl.when(gi >= 1)
                    def _():
                        wait_store(g - LOOKAHEAD, q)
                    issue_gather(g + LOOKAHEAD, q)
                else:
                    @pl.when(gi < n_groups - 1)
                    def _():
                        wait_store(g - LOOKAHEAD, q)
                        issue_gather(g + LOOKAHEAD, q)
            return carry

        lax.fori_loop(0, n_groups, group_body, 0)

        for p in range(NBUF):
            wait_store(n_chunks - NBUF + p, p)

    return gkern(idx_flat, table_lin)


def kernel(indices, table):
    B = indices.shape[0] * indices.shape[1]
    idx_flat = indices.reshape(B).astype(jnp.int32)

    tail_lin = (table[NBLK * CB:] * SCALE).reshape(VTAIL * D)
    tab_lin = _transpose_scale(table.T, tail_lin)
    out = _gather(idx_flat, tab_lin.reshape(V, D))
    return out.reshape(indices.shape[0], indices.shape[1], D)
